# baseline (speedup 1.0000x reference)
import functools

import jax
import jax.numpy as jnp
from jax import lax
from jax.experimental import pallas as pl
from jax.experimental.pallas import tpu as pltpu
from jax.experimental.pallas import tpu_sc as plsc

N = 10000
E = 320000
D = 128

NC = 2
NS = 16
NW = NC * NS
EPW = E // NW
CHUNK = 80
NCHUNK = EPW // CHUNK
ROWCH = 80
NROWCH = N // ROWCH
KMAX = (NROWCH + NS - 1) // NS
QR = 2


def _sc_agg_body(with_deg, *refs):
    if with_deg:
        (y, src_h, dst_h, z128, ones128, agg_out, deg_out,
         sbuf0, sbuf1, dbuf0, dbuf1,
         rows0, rows1, zrow, onesv, acc,
         gsem0, gsem1, ssem0, ssem1) = refs
    else:
        (y, src_h, dst_h, z128, agg_out,
         sbuf0, sbuf1, dbuf0, dbuf1,
         rows0, rows1, zrow, acc,
         gsem0, gsem1, ssem0, ssem1) = refs

    c = lax.axis_index("c")
    s = lax.axis_index("s")
    wid = c * NS + s

    pltpu.sync_copy(z128, zrow)
    if with_deg:
        pltpu.sync_copy(ones128, onesv)

    def zero_acc():
        def zbody(k, carry):
            ch = s + k * NS

            @pl.when(ch < NROWCH)
            def _():
                pltpu.sync_copy(zrow, acc.at[pl.ds(ch * ROWCH, ROWCH)])

            return carry

        lax.fori_loop(0, KMAX, zbody, 0)

    def writeback(out_ref):
        def wbody(k, carry):
            ch = s + k * NS

            @pl.when(ch < NROWCH)
            def _():
                r = pl.ds(ch * ROWCH, ROWCH)
                pltpu.sync_copy(acc.at[r], out_ref.at[c, r])

            return carry

        lax.fori_loop(0, KMAX, wbody, 0)

    base = wid * NCHUNK

    def load_idx(i, sb, db):
        off = pl.ds((base + i) * CHUNK, CHUNK)
        pltpu.sync_copy(src_h.at[off], sb)
        pltpu.sync_copy(dst_h.at[off], db)

    def load_didx(i, db):
        off = pl.ds((base + i) * CHUNK, CHUNK)
        pltpu.sync_copy(dst_h.at[off], db)

    def gather_start(sb, rw, sem):
        pltpu.async_copy(y.at[sb], rw, sem)

    def gather_wait(sb, rw, sem):
        pltpu.make_async_copy(y.at[sb], rw, sem).wait()

    sbuf = (sbuf0, sbuf1)
    dbuf = (dbuf0, dbuf1)
    rows = (rows0, rows1)
    gsem = (gsem0, gsem1)
    ssem = (ssem0, ssem1)

    zero_acc()
    plsc.subcore_barrier()

    for b in range(QR):
        load_idx(b, sbuf[b], dbuf[b])
        gather_start(sbuf[b], rows[b], gsem[b])

    def pbody(k, carry):
        scd = []
        for b in range(QR):
            gather_wait(sbuf[b], rows[b], gsem[b])
            scd.append(pltpu.async_copy(rows[b], acc.at[dbuf[b]],
                                        ssem[b], add=True))
        for b in range(QR):
            scd[b].wait()
            i = QR * k + QR + b

            @pl.when(i < NCHUNK)
            def _():
                load_idx(i, sbuf[b], dbuf[b])
                gather_start(sbuf[b], rows[b], gsem[b])

        return carry

    lax.fori_loop(0, NCHUNK // QR, pbody, 0)
    gather_wait(sbuf[0], rows[0], gsem[0])
    pltpu.sync_copy(rows[0], acc.at[dbuf[0]], add=True)

    plsc.subcore_barrier()
    writeback(agg_out)

    if with_deg:
        zero_acc()
        plsc.subcore_barrier()

        for b in range(QR):
            load_didx(b, dbuf[b])

        def dbody(k, carry):
            dd = []
            for b in range(QR):
                dd.append(pltpu.async_copy(onesv, acc.at[dbuf[b]],
                                           ssem[b], add=True))
            for b in range(QR):
                dd[b].wait()
                i = QR * k + QR + b

                @pl.when(i < NCHUNK)
                def _():
                    load_didx(i, dbuf[b])

            return carry

        lax.fori_loop(0, NCHUNK // QR, dbody, 0)
        pltpu.sync_copy(onesv, acc.at[dbuf[0]], add=True)
        plsc.subcore_barrier()
        writeback(deg_out)


def _make_sc_agg(with_deg):
    out_type = [jax.ShapeDtypeStruct((NC, N, D), jnp.float32)]
    scratch = (
        [pltpu.VMEM((CHUNK,), jnp.int32)] * QR
        + [pltpu.VMEM((CHUNK,), jnp.int32)] * QR
        + [pltpu.VMEM((CHUNK, D), jnp.float32)] * QR
        + [pltpu.VMEM((ROWCH, D), jnp.float32)]
    )
    if with_deg:
        out_type.append(jax.ShapeDtypeStruct((NC, N, D), jnp.float32))
        scratch.append(pltpu.VMEM((CHUNK, D), jnp.float32))
    scratch += (
        [pltpu.VMEM_SHARED((N, D), jnp.float32)]
        + [pltpu.SemaphoreType.DMA] * (2 * QR)
    )
    return pl.kernel(
        functools.partial(_sc_agg_body, with_deg),
        out_type=out_type,
        mesh=plsc.VectorSubcoreMesh(core_axis_name="c", subcore_axis_name="s"),
        scratch_types=scratch,
    )


_sc_agg_deg = _make_sc_agg(True)
_sc_agg = _make_sc_agg(False)


BLK = 1000
NBLK = N // BLK


def _mm2_body(x_ref, ws_ref, wn_ref, b_ref, xs_ref, y_ref):
    x = x_ref[...]
    xs_ref[...] = jnp.dot(x, ws_ref[...],
                          preferred_element_type=jnp.float32) + b_ref[...]
    y_ref[...] = jnp.dot(x, wn_ref[...], preferred_element_type=jnp.float32)


def _combine_mm2_body(xs_ref, a0_ref, a1_ref, d0_ref, d1_ref,
                      ws_ref, wn_ref, b_ref, xs2_ref, y2_ref):
    deg = d0_ref[...] + d1_ref[...]
    inv = 1.0 / jnp.maximum(deg, 1.0)
    h = xs_ref[...] + (a0_ref[...] + a1_ref[...]) * inv
    h = jnp.maximum(h, 0.0)
    xs2_ref[...] = jnp.dot(h, ws_ref[...],
                           preferred_element_type=jnp.float32) + b_ref[...]
    y2_ref[...] = jnp.dot(h, wn_ref[...], preferred_element_type=jnp.float32)


def _final_body(xs_ref, a0_ref, a1_ref, d0_ref, d1_ref, o_ref):
    deg = d0_ref[...] + d1_ref[...]
    inv = 1.0 / jnp.maximum(deg, 1.0)
    o_ref[...] = xs_ref[...] + (a0_ref[...] + a1_ref[...]) * inv


def _row_spec(width=D):
    return pl.BlockSpec((BLK, width), lambda i: (i, 0))


def _full_spec(shape):
    return pl.BlockSpec(shape, lambda i: tuple(0 for _ in shape))


def _mm2(x, ws, wn, b):
    return pl.pallas_call(
        _mm2_body,
        grid=(NBLK,),
        in_specs=[_row_spec(), _full_spec((D, D)), _full_spec((D, D)),
                  _full_spec((1, D))],
        out_specs=[_row_spec(), _row_spec()],
        out_shape=[jax.ShapeDtypeStruct((N, D), jnp.float32),
                   jax.ShapeDtypeStruct((N, D), jnp.float32)],
    )(x, ws, wn, b.reshape(1, D))


def _combine_mm2(xs, a0, a1, d0, d1, ws, wn, b):
    return pl.pallas_call(
        _combine_mm2_body,
        grid=(NBLK,),
        in_specs=[_row_spec(), _row_spec(), _row_spec(),
                  _row_spec(1), _row_spec(1),
                  _full_spec((D, D)), _full_spec((D, D)), _full_spec((1, D))],
        out_specs=[_row_spec(), _row_spec()],
        out_shape=[jax.ShapeDtypeStruct((N, D), jnp.float32),
                   jax.ShapeDtypeStruct((N, D), jnp.float32)],
    )(xs, a0, a1, d0, d1, ws, wn, b.reshape(1, D))


def _final(xs, a0, a1, d0, d1):
    return pl.pallas_call(
        _final_body,
        grid=(NBLK,),
        in_specs=[_row_spec(), _row_spec(), _row_spec(),
                  _row_spec(1), _row_spec(1)],
        out_specs=_row_spec(),
        out_shape=jax.ShapeDtypeStruct((N, D), jnp.float32),
    )(xs, a0, a1, d0, d1)


def kernel(node_features, edge_index, w_self1, w_neigh1, b1,
           w_self2, w_neigh2, b2):
    z128 = jnp.zeros((ROWCH, D), jnp.float32)
    ones128 = jnp.ones((CHUNK, D), jnp.float32)

    src = edge_index[0]
    dst = edge_index[1]

    xs1, y1 = _mm2(node_features, w_self1, w_neigh1, b1)
    agg1, degf = _sc_agg_deg(y1, src, dst, z128, ones128)
    d0 = degf[0, :, :1]
    d1 = degf[1, :, :1]
    xs2, y2 = _combine_mm2(xs1, agg1[0], agg1[1], d0, d1,
                           w_self2, w_neigh2, b2)
    (agg2,) = _sc_agg(y2, src, dst, z128)
    return _final(xs2, agg2[0], agg2[1], d0, d1)


# speedup vs baseline: 1.0353x; 1.0353x over previous
import functools

import jax
import jax.numpy as jnp
from jax import lax
from jax.experimental import pallas as pl
from jax.experimental.pallas import tpu as pltpu
from jax.experimental.pallas import tpu_sc as plsc

N = 10000
E = 320000
D = 128

NC = 2
NS = 16
NW = NC * NS
EPW = E // NW
CHUNK = 80
NCHUNK = EPW // CHUNK
ROWCH = 80
NROWCH = N // ROWCH
KMAX = (NROWCH + NS - 1) // NS


def _sc_agg_body(with_deg, *refs):
    if with_deg:
        (y, src_h, dst_h, z128, ones128, agg_out, deg_out,
         sbuf0, sbuf1, dbuf0, dbuf1, rows0, rows1, zrow, onesv, acc,
         gsem0, gsem1) = refs
    else:
        (y, src_h, dst_h, z128, agg_out,
         sbuf0, sbuf1, dbuf0, dbuf1, rows0, rows1, zrow, acc,
         gsem0, gsem1) = refs

    c = lax.axis_index("c")
    s = lax.axis_index("s")
    wid = c * NS + s

    pltpu.sync_copy(z128, zrow)
    if with_deg:
        pltpu.sync_copy(ones128, onesv)

    def zero_acc():
        def zbody(k, carry):
            ch = s + k * NS

            @pl.when(ch < NROWCH)
            def _():
                pltpu.sync_copy(zrow, acc.at[pl.ds(ch * ROWCH, ROWCH)])

            return carry

        lax.fori_loop(0, KMAX, zbody, 0)

    def writeback(out_ref):
        def wbody(k, carry):
            ch = s + k * NS

            @pl.when(ch < NROWCH)
            def _():
                r = pl.ds(ch * ROWCH, ROWCH)
                pltpu.sync_copy(acc.at[r], out_ref.at[c, r])

            return carry

        lax.fori_loop(0, KMAX, wbody, 0)

    base = wid * NCHUNK

    def load_idx(i, sbuf, dbuf):
        off = pl.ds((base + i) * CHUNK, CHUNK)
        pltpu.sync_copy(src_h.at[off], sbuf)
        pltpu.sync_copy(dst_h.at[off], dbuf)

    def load_didx(i, dbuf):
        off = pl.ds((base + i) * CHUNK, CHUNK)
        pltpu.sync_copy(dst_h.at[off], dbuf)

    def gather_start(sbuf, rows, sem):
        pltpu.async_copy(y.at[sbuf], rows, sem)

    def gather_wait(sbuf, rows, sem):
        pltpu.make_async_copy(y.at[sbuf], rows, sem).wait()

    zero_acc()
    plsc.subcore_barrier()

    load_idx(0, sbuf0, dbuf0)
    gather_start(sbuf0, rows0, gsem0)

    def pbody(k, carry):
        i0 = 2 * k
        load_idx(i0 + 1, sbuf1, dbuf1)
        gather_start(sbuf1, rows1, gsem1)
        gather_wait(sbuf0, rows0, gsem0)
        pltpu.sync_copy(rows0, acc.at[dbuf0], add=True)
        load_idx(i0 + 2, sbuf0, dbuf0)
        gather_start(sbuf0, rows0, gsem0)
        gather_wait(sbuf1, rows1, gsem1)
        pltpu.sync_copy(rows1, acc.at[dbuf1], add=True)
        return carry

    lax.fori_loop(0, (NCHUNK - 1) // 2, pbody, 0)
    gather_wait(sbuf0, rows0, gsem0)
    pltpu.sync_copy(rows0, acc.at[dbuf0], add=True)

    plsc.subcore_barrier()
    writeback(agg_out)

    if with_deg:
        zero_acc()
        plsc.subcore_barrier()

        def dbody(i, carry):
            load_didx(i, dbuf0)
            pltpu.sync_copy(onesv, acc.at[dbuf0], add=True)
            return carry

        lax.fori_loop(0, NCHUNK, dbody, 0)
        plsc.subcore_barrier()
        writeback(deg_out)


def _make_sc_agg(with_deg):
    out_type = [jax.ShapeDtypeStruct((NC, N, D), jnp.float32)]
    scratch = [
        pltpu.VMEM((CHUNK,), jnp.int32),
        pltpu.VMEM((CHUNK,), jnp.int32),
        pltpu.VMEM((CHUNK,), jnp.int32),
        pltpu.VMEM((CHUNK,), jnp.int32),
        pltpu.VMEM((CHUNK, D), jnp.float32),
        pltpu.VMEM((CHUNK, D), jnp.float32),
        pltpu.VMEM((ROWCH, D), jnp.float32),
    ]
    if with_deg:
        out_type.append(jax.ShapeDtypeStruct((NC, N, D), jnp.float32))
        scratch.append(pltpu.VMEM((CHUNK, D), jnp.float32))
    scratch += [
        pltpu.VMEM_SHARED((N, D), jnp.float32),
        pltpu.SemaphoreType.DMA,
        pltpu.SemaphoreType.DMA,
    ]
    return pl.kernel(
        functools.partial(_sc_agg_body, with_deg),
        out_type=out_type,
        mesh=plsc.VectorSubcoreMesh(core_axis_name="c", subcore_axis_name="s"),
        scratch_types=scratch,
    )


_sc_agg_deg = _make_sc_agg(True)
_sc_agg = _make_sc_agg(False)


BLK = 1000
NBLK = N // BLK


def _mm2_body(x_ref, ws_ref, wn_ref, b_ref, xs_ref, y_ref):
    x = x_ref[...]
    xs_ref[...] = jnp.dot(x, ws_ref[...],
                          preferred_element_type=jnp.float32) + b_ref[...]
    y_ref[...] = jnp.dot(x, wn_ref[...], preferred_element_type=jnp.float32)


def _combine_mm2_body(xs_ref, a0_ref, a1_ref, d0_ref, d1_ref,
                      ws_ref, wn_ref, b_ref, xs2_ref, y2_ref):
    deg = d0_ref[...] + d1_ref[...]
    inv = 1.0 / jnp.maximum(deg, 1.0)
    h = xs_ref[...] + (a0_ref[...] + a1_ref[...]) * inv
    h = jnp.maximum(h, 0.0)
    xs2_ref[...] = jnp.dot(h, ws_ref[...],
                           preferred_element_type=jnp.float32) + b_ref[...]
    y2_ref[...] = jnp.dot(h, wn_ref[...], preferred_element_type=jnp.float32)


def _final_body(xs_ref, a0_ref, a1_ref, d0_ref, d1_ref, o_ref):
    deg = d0_ref[...] + d1_ref[...]
    inv = 1.0 / jnp.maximum(deg, 1.0)
    o_ref[...] = xs_ref[...] + (a0_ref[...] + a1_ref[...]) * inv


def _row_spec(width=D):
    return pl.BlockSpec((BLK, width), lambda i: (i, 0))


def _full_spec(shape):
    return pl.BlockSpec(shape, lambda i: tuple(0 for _ in shape))


def _mm2(x, ws, wn, b):
    return pl.pallas_call(
        _mm2_body,
        grid=(NBLK,),
        in_specs=[_row_spec(), _full_spec((D, D)), _full_spec((D, D)),
                  _full_spec((1, D))],
        out_specs=[_row_spec(), _row_spec()],
        out_shape=[jax.ShapeDtypeStruct((N, D), jnp.float32),
                   jax.ShapeDtypeStruct((N, D), jnp.float32)],
    )(x, ws, wn, b.reshape(1, D))


def _combine_mm2(xs, a0, a1, d0, d1, ws, wn, b):
    return pl.pallas_call(
        _combine_mm2_body,
        grid=(NBLK,),
        in_specs=[_row_spec(), _row_spec(), _row_spec(),
                  _row_spec(1), _row_spec(1),
                  _full_spec((D, D)), _full_spec((D, D)), _full_spec((1, D))],
        out_specs=[_row_spec(), _row_spec()],
        out_shape=[jax.ShapeDtypeStruct((N, D), jnp.float32),
                   jax.ShapeDtypeStruct((N, D), jnp.float32)],
    )(xs, a0, a1, d0, d1, ws, wn, b.reshape(1, D))


def _final(xs, a0, a1, d0, d1):
    return pl.pallas_call(
        _final_body,
        grid=(NBLK,),
        in_specs=[_row_spec(), _row_spec(), _row_spec(),
                  _row_spec(1), _row_spec(1)],
        out_specs=_row_spec(),
        out_shape=jax.ShapeDtypeStruct((N, D), jnp.float32),
    )(xs, a0, a1, d0, d1)


def kernel(node_features, edge_index, w_self1, w_neigh1, b1,
           w_self2, w_neigh2, b2):
    z128 = jnp.zeros((ROWCH, D), jnp.float32)
    ones128 = jnp.ones((CHUNK, D), jnp.float32)

    src = edge_index[0]
    dst = edge_index[1]

    xs1, y1 = _mm2(node_features, w_self1, w_neigh1, b1)
    agg1, degf = _sc_agg_deg(y1, src, dst, z128, ones128)
    d0 = degf[0, :, :1]
    d1 = degf[1, :, :1]
    xs2, y2 = _combine_mm2(xs1, agg1[0], agg1[1], d0, d1,
                           w_self2, w_neigh2, b2)
    (agg2,) = _sc_agg(y2, src, dst, z128)
    return _final(xs2, agg2[0], agg2[1], d0, d1)


# speedup vs baseline: 1.0425x; 1.0070x over previous
import functools

import jax
import jax.numpy as jnp
from jax import lax
from jax.experimental import pallas as pl
from jax.experimental.pallas import tpu as pltpu
from jax.experimental.pallas import tpu_sc as plsc

N = 10000
E = 320000
D = 128

NC = 2
NS = 16
NW = NC * NS
EPW = E // NW
CHUNK = 80
NCHUNK = EPW // CHUNK
ROWCH = 80
NROWCH = N // ROWCH
KMAX = (NROWCH + NS - 1) // NS


def _sc_agg_body(with_deg, *refs):
    if with_deg:
        (y, src_h, dst_h, z128, ones128, agg_out, deg_out,
         sbuf0, sbuf1, dbuf0, dbuf1, rows0, rows1, zrow, onesv, acc,
         gsem0, gsem1) = refs
    else:
        (y, src_h, dst_h, z128, agg_out,
         sbuf0, sbuf1, dbuf0, dbuf1, rows0, rows1, zrow, acc,
         gsem0, gsem1) = refs

    c = lax.axis_index("c")
    s = lax.axis_index("s")
    wid = c * NS + s

    pltpu.sync_copy(z128, zrow)
    if with_deg:
        pltpu.sync_copy(ones128, onesv)

    def zero_acc():
        def zbody(k, carry):
            ch = s + k * NS

            @pl.when(ch < NROWCH)
            def _():
                pltpu.sync_copy(zrow, acc.at[pl.ds(ch * ROWCH, ROWCH)])

            return carry

        lax.fori_loop(0, KMAX, zbody, 0)

    def writeback(out_ref):
        def wbody(k, carry):
            ch = s + k * NS

            @pl.when(ch < NROWCH)
            def _():
                r = pl.ds(ch * ROWCH, ROWCH)
                pltpu.sync_copy(acc.at[r], out_ref.at[c, r])

            return carry

        lax.fori_loop(0, KMAX, wbody, 0)

    base = wid * NCHUNK

    def load_idx(i, sbuf, dbuf):
        off = pl.ds((base + i) * CHUNK, CHUNK)
        pltpu.sync_copy(src_h.at[off], sbuf)
        pltpu.sync_copy(dst_h.at[off], dbuf)

    def load_didx(i, dbuf):
        off = pl.ds((base + i) * CHUNK, CHUNK)
        pltpu.sync_copy(dst_h.at[off], dbuf)

    def gather_start(sbuf, rows, sem):
        pltpu.async_copy(y.at[sbuf], rows, sem)

    def gather_wait(sbuf, rows, sem):
        pltpu.make_async_copy(y.at[sbuf], rows, sem).wait()

    zero_acc()
    plsc.subcore_barrier()

    load_idx(0, sbuf0, dbuf0)
    gather_start(sbuf0, rows0, gsem0)

    def pbody(k, carry):
        i0 = 2 * k
        load_idx(i0 + 1, sbuf1, dbuf1)
        gather_start(sbuf1, rows1, gsem1)
        gather_wait(sbuf0, rows0, gsem0)
        pltpu.sync_copy(rows0, acc.at[dbuf0], add=True)
        load_idx(i0 + 2, sbuf0, dbuf0)
        gather_start(sbuf0, rows0, gsem0)
        gather_wait(sbuf1, rows1, gsem1)
        pltpu.sync_copy(rows1, acc.at[dbuf1], add=True)
        return carry

    lax.fori_loop(0, (NCHUNK - 1) // 2, pbody, 0)
    gather_wait(sbuf0, rows0, gsem0)
    pltpu.sync_copy(rows0, acc.at[dbuf0], add=True)

    plsc.subcore_barrier()
    writeback(agg_out)

    if with_deg:
        zero_acc()
        plsc.subcore_barrier()

        load_didx(0, dbuf0)
        load_didx(1, dbuf1)

        def dbody(k, carry):
            i0 = 2 * k
            d0 = pltpu.async_copy(onesv, acc.at[dbuf0], gsem0, add=True)
            pltpu.sync_copy(onesv, acc.at[dbuf1], add=True)
            d0.wait()

            @pl.when(i0 + 2 < NCHUNK)
            def _():
                load_didx(i0 + 2, dbuf0)

            @pl.when(i0 + 3 < NCHUNK)
            def _():
                load_didx(i0 + 3, dbuf1)

            return carry

        lax.fori_loop(0, NCHUNK // 2, dbody, 0)
        pltpu.sync_copy(onesv, acc.at[dbuf0], add=True)
        plsc.subcore_barrier()
        writeback(deg_out)


def _make_sc_agg(with_deg):
    out_type = [jax.ShapeDtypeStruct((NC, N, D), jnp.float32)]
    scratch = [
        pltpu.VMEM((CHUNK,), jnp.int32),
        pltpu.VMEM((CHUNK,), jnp.int32),
        pltpu.VMEM((CHUNK,), jnp.int32),
        pltpu.VMEM((CHUNK,), jnp.int32),
        pltpu.VMEM((CHUNK, D), jnp.float32),
        pltpu.VMEM((CHUNK, D), jnp.float32),
        pltpu.VMEM((ROWCH, D), jnp.float32),
    ]
    if with_deg:
        out_type.append(jax.ShapeDtypeStruct((NC, N, D), jnp.float32))
        scratch.append(pltpu.VMEM((CHUNK, D), jnp.float32))
    scratch += [
        pltpu.VMEM_SHARED((N, D), jnp.float32),
        pltpu.SemaphoreType.DMA,
        pltpu.SemaphoreType.DMA,
    ]
    return pl.kernel(
        functools.partial(_sc_agg_body, with_deg),
        out_type=out_type,
        mesh=plsc.VectorSubcoreMesh(core_axis_name="c", subcore_axis_name="s"),
        scratch_types=scratch,
    )


_sc_agg_deg = _make_sc_agg(True)
_sc_agg = _make_sc_agg(False)


BLK = 1000
NBLK = N // BLK


def _mm2_body(x_ref, ws_ref, wn_ref, b_ref, xs_ref, y_ref):
    x = x_ref[...]
    xs_ref[...] = jnp.dot(x, ws_ref[...],
                          preferred_element_type=jnp.float32) + b_ref[...]
    y_ref[...] = jnp.dot(x, wn_ref[...], preferred_element_type=jnp.float32)


def _combine_mm2_body(xs_ref, a0_ref, a1_ref, d0_ref, d1_ref,
                      ws_ref, wn_ref, b_ref, xs2_ref, y2_ref):
    deg = d0_ref[...] + d1_ref[...]
    inv = 1.0 / jnp.maximum(deg, 1.0)
    h = xs_ref[...] + (a0_ref[...] + a1_ref[...]) * inv
    h = jnp.maximum(h, 0.0)
    xs2_ref[...] = jnp.dot(h, ws_ref[...],
                           preferred_element_type=jnp.float32) + b_ref[...]
    y2_ref[...] = jnp.dot(h, wn_ref[...], preferred_element_type=jnp.float32)


def _final_body(xs_ref, a0_ref, a1_ref, d0_ref, d1_ref, o_ref):
    deg = d0_ref[...] + d1_ref[...]
    inv = 1.0 / jnp.maximum(deg, 1.0)
    o_ref[...] = xs_ref[...] + (a0_ref[...] + a1_ref[...]) * inv


def _row_spec(width=D):
    return pl.BlockSpec((BLK, width), lambda i: (i, 0))


def _full_spec(shape):
    return pl.BlockSpec(shape, lambda i: tuple(0 for _ in shape))


def _mm2(x, ws, wn, b):
    return pl.pallas_call(
        _mm2_body,
        grid=(NBLK,),
        in_specs=[_row_spec(), _full_spec((D, D)), _full_spec((D, D)),
                  _full_spec((1, D))],
        out_specs=[_row_spec(), _row_spec()],
        out_shape=[jax.ShapeDtypeStruct((N, D), jnp.float32),
                   jax.ShapeDtypeStruct((N, D), jnp.float32)],
    )(x, ws, wn, b.reshape(1, D))


def _combine_mm2(xs, a0, a1, d0, d1, ws, wn, b):
    return pl.pallas_call(
        _combine_mm2_body,
        grid=(NBLK,),
        in_specs=[_row_spec(), _row_spec(), _row_spec(),
                  _row_spec(1), _row_spec(1),
                  _full_spec((D, D)), _full_spec((D, D)), _full_spec((1, D))],
        out_specs=[_row_spec(), _row_spec()],
        out_shape=[jax.ShapeDtypeStruct((N, D), jnp.float32),
                   jax.ShapeDtypeStruct((N, D), jnp.float32)],
    )(xs, a0, a1, d0, d1, ws, wn, b.reshape(1, D))


def _final(xs, a0, a1, d0, d1):
    return pl.pallas_call(
        _final_body,
        grid=(NBLK,),
        in_specs=[_row_spec(), _row_spec(), _row_spec(),
                  _row_spec(1), _row_spec(1)],
        out_specs=_row_spec(),
        out_shape=jax.ShapeDtypeStruct((N, D), jnp.float32),
    )(xs, a0, a1, d0, d1)


def kernel(node_features, edge_index, w_self1, w_neigh1, b1,
           w_self2, w_neigh2, b2):
    z128 = jnp.zeros((ROWCH, D), jnp.float32)
    ones128 = jnp.ones((CHUNK, D), jnp.float32)

    src = edge_index[0]
    dst = edge_index[1]

    xs1, y1 = _mm2(node_features, w_self1, w_neigh1, b1)
    agg1, degf = _sc_agg_deg(y1, src, dst, z128, ones128)
    d0 = degf[0, :, :1]
    d1 = degf[1, :, :1]
    xs2, y2 = _combine_mm2(xs1, agg1[0], agg1[1], d0, d1,
                           w_self2, w_neigh2, b2)
    (agg2,) = _sc_agg(y2, src, dst, z128)
    return _final(xs2, agg2[0], agg2[1], d0, d1)
